# Initial kernel scaffold; baseline (speedup 1.0000x reference)
import jax
import jax.numpy as jnp
from jax.experimental import pallas as pl


def kernel(xyz_1, xyz_2, points_1, points_2, W1, b1, W2, b2):
    raise NotImplementedError("write your pallas kernel here")


# speedup vs baseline: 40.8164x; 40.8164x over previous
import functools

import jax
import jax.numpy as jnp
from jax.experimental import pallas as pl

_TN = 512


def _body(xyz1_ref, xyz2t_ref, p1_ref, p2_ref, w1_ref, b1_ref, w2_ref, b2_ref,
          out_ref, *, m, k_nn):
    x1 = xyz1_ref[0]
    x2 = xyz2t_ref[0]

    d = None
    for c in range(x1.shape[0]):
        diff = x2[:, c:c + 1] - x1[c:c + 1, :]
        sq = diff * diff
        d = sq if d is None else d + sq

    iota = jax.lax.broadcasted_iota(jnp.int32, d.shape, 1)
    big = jnp.float32(3.0e38)

    d_ks, i_ks = [], []
    dd = d
    for _ in range(k_nn):
        dmin = jnp.min(dd, axis=1, keepdims=True)
        sel = dd == dmin
        idx = jnp.min(jnp.where(sel, iota, jnp.int32(m)), axis=1,
                      keepdims=True)
        d_ks.append(dmin)
        i_ks.append(idx)
        dd = jnp.where(iota == idx, big, dd)

    recips = [1.0 / (dk + jnp.float32(1e-8)) for dk in d_ks]
    norm = functools.reduce(lambda a, b: a + b, recips)
    weights = [r / norm for r in recips]

    s = None
    for ik, wk in zip(i_ks, weights):
        term = jnp.where(iota == ik, wk, jnp.float32(0.0))
        s = term if s is None else s + term

    dn = (((1,), (0,)), ((), ()))
    p1 = jax.lax.dot_general(w1_ref[...], p1_ref[0], dn,
                             preferred_element_type=jnp.float32) + b1_ref[...]
    p2 = jax.lax.dot_general(w2_ref[...], p2_ref[0], dn,
                             preferred_element_type=jnp.float32) + b2_ref[...]
    dn_nt = (((1,), (1,)), ((), ()))
    interp_t = jax.lax.dot_general(p1, s, dn_nt,
                                   preferred_element_type=jnp.float32)
    out_ref[0] = interp_t + p2


def kernel(xyz_1, xyz_2, points_1, points_2, W1, b1, W2, b2):
    b, _, m = xyz_1.shape
    n = xyz_2.shape[2]
    c_in = points_1.shape[1]
    c_out = points_2.shape[1]
    tn = min(_TN, n)
    nt = n // tn

    xyz2t = jnp.transpose(xyz_2, (0, 2, 1))
    b1c = b1[:, None]
    b2c = b2[:, None]

    out = pl.pallas_call(
        functools.partial(_body, m=m, k_nn=3),
        grid=(b, nt),
        in_specs=[
            pl.BlockSpec((1, 3, m), lambda bi, ti: (bi, 0, 0)),
            pl.BlockSpec((1, tn, 3), lambda bi, ti: (bi, ti, 0)),
            pl.BlockSpec((1, c_in, m), lambda bi, ti: (bi, 0, 0)),
            pl.BlockSpec((1, c_out, tn), lambda bi, ti: (bi, 0, ti)),
            pl.BlockSpec((c_out, c_in), lambda bi, ti: (0, 0)),
            pl.BlockSpec((c_out, 1), lambda bi, ti: (0, 0)),
            pl.BlockSpec((c_out, c_out), lambda bi, ti: (0, 0)),
            pl.BlockSpec((c_out, 1), lambda bi, ti: (0, 0)),
        ],
        out_specs=pl.BlockSpec((1, c_out, tn), lambda bi, ti: (bi, 0, ti)),
        out_shape=jax.ShapeDtypeStruct((b, c_out, n), jnp.float32),
    )(xyz_1, xyz2t, points_1, points_2, W1, b1c, W2, b2c)
    return (xyz_2, out)
